# Initial kernel scaffold; baseline (speedup 1.0000x reference)
import jax
import jax.numpy as jnp
from jax.experimental import pallas as pl


def kernel(all_timestamps, ts_w, pos_w):
    raise NotImplementedError("write your pallas kernel here")


# speedup vs baseline: 375.7658x; 375.7658x over previous
import functools

import jax
import jax.numpy as jnp
from jax.experimental import pallas as pl

_N = 200


def _body(col_ref, row_ref, pos_ref, tsw_ref, out_ref):
    c = col_ref[0]
    r = row_ref[0]
    diff = c - r
    mag = jnp.maximum(jnp.abs(diff), 1).astype(jnp.float32)
    b = (jnp.log(mag) / 0.301).astype(jnp.int32)
    b = jnp.clip(b, 0, 127)
    tsw = jnp.broadcast_to(tsw_ref[...], (_N, 128))
    vals = jnp.take_along_axis(tsw, b, axis=1)
    out_ref[0] = vals + pos_ref[...]


@functools.partial(jax.jit, static_argnames=())
def kernel(all_timestamps, ts_w, pos_w):
    N = _N
    B = all_timestamps.shape[0]
    ts = all_timestamps
    nxt = jnp.concatenate([ts[:, 1:], ts[:, N - 1 : N]], axis=1)
    col = nxt.reshape(B, N, 1)
    row = ts.reshape(B, 1, N)
    idx = jnp.arange(N)
    pos_mat = pos_w[idx[None, :] - idx[:, None] + N - 1].astype(jnp.float32)
    tsw = ts_w[:128].reshape(1, 128)

    out = pl.pallas_call(
        _body,
        grid=(B,),
        in_specs=[
            pl.BlockSpec((1, N, 1), lambda b: (b, 0, 0)),
            pl.BlockSpec((1, 1, N), lambda b: (b, 0, 0)),
            pl.BlockSpec((N, N), lambda b: (0, 0)),
            pl.BlockSpec((1, 128), lambda b: (0, 0)),
        ],
        out_specs=pl.BlockSpec((1, N, N), lambda b: (b, 0, 0)),
        out_shape=jax.ShapeDtypeStruct((B, N, N), jnp.float32),
    )(col, row, pos_mat, tsw)
    return out
